# baseline (speedup 1.0000x reference)
import jax
import jax.numpy as jnp
from jax import lax
from jax.experimental import pallas as pl
from jax.experimental.pallas import tpu as pltpu
from jax.experimental.pallas import tpu_sc as plsc

_NC = 2
_NS = 16
_NW = _NC * _NS


def _gather_kernel(b_total: int, d: int, chunk: int, nbuf: int):
    b_per_w = b_total // _NW
    n_chunks = b_per_w // chunk
    n_groups = n_chunks // nbuf
    mesh = plsc.VectorSubcoreMesh(core_axis_name="c", subcore_axis_name="s")

    def body(idx_hbm, table_hbm, out_hbm, idx_v, *scratch):
        bufs = scratch[:nbuf]
        gsem, wsem = scratch[nbuf], scratch[nbuf + 1]
        wid = lax.axis_index("s") * _NC + lax.axis_index("c")
        base = wid * b_per_w
        pltpu.sync_copy(idx_hbm.at[wid], idx_v)

        def gather(j, b):
            return pltpu.make_async_copy(
                table_hbm.at[idx_v.at[j]], bufs[b], gsem)

        def write(j, b):
            return pltpu.make_async_copy(
                bufs[b], out_hbm.at[pl.ds(base + j * chunk, chunk)], wsem)

        for b in range(nbuf):
            gather(b, b).start()

        def group(g, carry):
            j0 = g * nbuf
            for b in range(nbuf):
                gather(j0 + b, b).wait()
            for b in range(nbuf):
                write(j0 + b, b).start()
            for b in range(nbuf):
                write(j0 + b, b).wait()
            for b in range(nbuf):
                gather(j0 + nbuf + b, b).start()
            return carry

        lax.fori_loop(0, n_groups - 1, group, 0, unroll=False)

        j0 = (n_groups - 1) * nbuf
        for b in range(nbuf):
            gather(j0 + b, b).wait()
        for b in range(nbuf):
            write(j0 + b, b).start()
        for b in range(nbuf):
            write(j0 + b, b).wait()

    return pl.kernel(
        body,
        out_type=jax.ShapeDtypeStruct((b_total, d), jnp.float32),
        mesh=mesh,
        scratch_types=(
            [pltpu.VMEM((n_chunks, chunk), jnp.int32)]
            + [pltpu.VMEM((chunk, d), jnp.float32) for _ in range(nbuf)]
            + [pltpu.SemaphoreType.DMA, pltpu.SemaphoreType.DMA]
        ),
    )


def kernel(x, table):
    batch, hist = x.shape
    vocab, d = table.shape
    b_total = batch * hist
    chunk = 128
    nbuf = 5
    idx3 = x.reshape(_NW, (b_total // _NW) // chunk, chunk)
    out = _gather_kernel(b_total, d, chunk, nbuf)(idx3, table)
    return out.reshape(batch, hist, d)


# speedup vs baseline: 1.5969x; 1.5969x over previous
import jax
import jax.numpy as jnp
from jax import lax
from jax.experimental import pallas as pl
from jax.experimental.pallas import tpu as pltpu
from jax.experimental.pallas import tpu_sc as plsc

_NC = 2
_NS = 16
_NW = _NC * _NS


def _gather_kernel(batch: int, hist: int, d: int, nbuf: int):
    nb = batch // _NW
    n_groups = nb // nbuf
    mesh = plsc.VectorSubcoreMesh(core_axis_name="c", subcore_axis_name="s")

    def body(idx_hbm, table_hbm, out_hbm, idx_v, *scratch):
        bufs = scratch[:nbuf]
        gsem, wsem = scratch[nbuf], scratch[nbuf + 1]
        wid = lax.axis_index("s") * _NC + lax.axis_index("c")
        base = wid * nb
        pltpu.sync_copy(idx_hbm.at[wid], idx_v)

        def gather(j, b):
            return pltpu.make_async_copy(
                table_hbm.at[idx_v.at[j]], bufs[b], gsem)

        def write(j, b):
            return pltpu.make_async_copy(
                bufs[b], out_hbm.at[base + j], wsem)

        for b in range(nbuf):
            gather(b, b).start()

        def group(g, carry):
            j0 = g * nbuf
            for b in range(nbuf):
                gather(j0 + b, b).wait()
            for b in range(nbuf):
                write(j0 + b, b).start()
            for b in range(nbuf):
                write(j0 + b, b).wait()
            for b in range(nbuf):
                gather(j0 + nbuf + b, b).start()
            return carry

        lax.fori_loop(0, n_groups - 1, group, 0, unroll=False)

        j0 = (n_groups - 1) * nbuf
        for b in range(nbuf):
            gather(j0 + b, b).wait()
        for b in range(nbuf):
            write(j0 + b, b).start()
        for b in range(nbuf):
            write(j0 + b, b).wait()

    return pl.kernel(
        body,
        out_type=jax.ShapeDtypeStruct((batch, hist, d), jnp.float32),
        mesh=mesh,
        scratch_types=(
            [pltpu.VMEM((nb, hist), jnp.int32)]
            + [pltpu.VMEM((hist, d), jnp.float32) for _ in range(nbuf)]
            + [pltpu.SemaphoreType.DMA, pltpu.SemaphoreType.DMA]
        ),
    )


def kernel(x, table):
    batch, hist = x.shape
    vocab, d = table.shape
    nbuf = 8
    idx3 = x.reshape(_NW, batch // _NW, hist)
    return _gather_kernel(batch, hist, d, nbuf)(idx3, table)
